# baseline (speedup 1.0000x reference)
import functools

import jax
import jax.numpy as jnp
from jax import lax
from jax.experimental import pallas as pl
from jax.experimental.pallas import tpu as pltpu
from jax.experimental.pallas import tpu_sc as plsc


def _gelu(x):
    return 0.5 * x * (1.0 + lax.erf(x * 0.7071067811865476))


_N = 10000
_D = 128
_NPAD = 10112
_NSUB = 16
_CHUNK = 128


def _spmm_call(table, idx_packed, n_chunks):
    rpt = _NPAD // _NSUB
    nt = n_chunks // 3

    def body(table_hbm, idx_hbm, zero_hbm, out_hbm,
             i0, i1, i2, r0, r1, r2, acc_sh, is0, is1, is2, g0, g1, g2):
        c = lax.axis_index("c")
        s = lax.axis_index("s")
        idx = (i0, i1, i2)
        rows = (r0, r1, r2)
        isem = (is0, is1, is2)
        gsem = (g0, g1, g2)
        pltpu.sync_copy(zero_hbm, acc_sh.at[pl.ds(s * rpt, rpt)])
        plsc.subcore_barrier()

        row0 = s * n_chunks

        def idx_load(k, u):
            pltpu.async_copy(idx_hbm.at[c, row0 + k], idx[u], isem[u])

        def idx_wait(k, u):
            pltpu.make_async_copy(idx_hbm.at[c, row0 + k], idx[u],
                                  isem[u]).wait()

        def gth(u):
            pltpu.async_copy(table_hbm.at[idx[u].at[0]], rows[u], gsem[u])

        def gth_wait(u):
            pltpu.make_async_copy(table_hbm.at[idx[u].at[0]], rows[u],
                                  gsem[u]).wait()

        for u in range(3):
            idx_load(u, u)
        idx_wait(0, 0)
        gth(0)
        idx_wait(1, 1)
        gth(1)

        def step(t, carry):
            for u in range(3):
                k = 3 * t + u
                u2 = (u + 2) % 3
                gth_wait(u)

                @pl.when(k + 2 < n_chunks)
                def _():
                    idx_wait(k + 2, u2)
                    gth(u2)

                pltpu.sync_copy(rows[u], acc_sh.at[idx[u].at[1]], add=True)

                @pl.when(k + 3 < n_chunks)
                def _():
                    idx_load(k + 3, u)
            return carry

        lax.fori_loop(0, nt, step, 0)
        plsc.subcore_barrier()
        pltpu.sync_copy(acc_sh.at[pl.ds(s * rpt, rpt)],
                        out_hbm.at[c, pl.ds(s * rpt, rpt)])

    mesh = plsc.VectorSubcoreMesh(core_axis_name="c", subcore_axis_name="s")
    f = pl.kernel(
        body,
        out_type=jax.ShapeDtypeStruct((2, _NPAD, _D), jnp.float32),
        mesh=mesh,
        scratch_types=[
            pltpu.VMEM((2, _CHUNK), jnp.int32),
            pltpu.VMEM((2, _CHUNK), jnp.int32),
            pltpu.VMEM((2, _CHUNK), jnp.int32),
            pltpu.VMEM((_CHUNK, _D), jnp.float32),
            pltpu.VMEM((_CHUNK, _D), jnp.float32),
            pltpu.VMEM((_CHUNK, _D), jnp.float32),
            pltpu.VMEM_SHARED((_NPAD, _D), jnp.float32),
        ] + [pltpu.SemaphoreType.DMA] * 6,
    )
    zero = jnp.zeros((rpt, _D), jnp.float32)
    return f(table, idx_packed, zero)


def _tc_mm(x, w):
    m = x.shape[0]
    bm = m // 8
    assert m % 8 == 0 and bm % 8 == 0

    def body(x_ref, w_ref, o_ref):
        o_ref[...] = jnp.dot(x_ref[...], w_ref[...],
                             preferred_element_type=jnp.float32)

    return pl.pallas_call(
        body,
        grid=(8,),
        in_specs=[
            pl.BlockSpec((bm, _D), lambda i: (i, 0)),
            pl.BlockSpec((_D, _D), lambda i: (0, 0)),
        ],
        out_specs=pl.BlockSpec((bm, _D), lambda i: (i, 0)),
        out_shape=jax.ShapeDtypeStruct((m, _D), jnp.float32),
    )(x, w)


def _tc_mm2(sp, w, b):
    m = sp.shape[1]
    bm = m // 8
    assert m % 8 == 0 and bm % 8 == 0

    def body(s_ref, w_ref, b_ref, o_ref):
        x = _gelu(s_ref[0] + s_ref[1] + b_ref[...])
        o_ref[...] = jnp.dot(x, w_ref[...], preferred_element_type=jnp.float32)

    return pl.pallas_call(
        body,
        grid=(8,),
        in_specs=[
            pl.BlockSpec((2, bm, _D), lambda i: (0, i, 0)),
            pl.BlockSpec((_D, _D), lambda i: (0, 0)),
            pl.BlockSpec((1, _D), lambda i: (0, 0)),
        ],
        out_specs=pl.BlockSpec((bm, _D), lambda i: (i, 0)),
        out_shape=jax.ShapeDtypeStruct((m, _D), jnp.float32),
    )(sp, w, b.reshape(1, _D))


def _tc_std(sp, b):
    def body(s_ref, b_ref, o_ref):
        x = _gelu(s_ref[0] + s_ref[1] + b_ref[...])
        mu = jnp.mean(x, axis=0, keepdims=True)
        xc = x - mu
        var = jnp.sum(xc * xc, axis=0, keepdims=True) / (_N - 1)
        o_ref[...] = xc * lax.rsqrt(var)

    return pl.pallas_call(
        body,
        grid=(1,),
        in_specs=[
            pl.BlockSpec((2, _N, _D), lambda i: (0, 0, 0)),
            pl.BlockSpec((1, _D), lambda i: (0, 0)),
        ],
        out_specs=pl.BlockSpec((_N, _D), lambda i: (0, 0)),
        out_shape=jax.ShapeDtypeStruct((_N, _D), jnp.float32),
    )(sp, b.reshape(1, _D))


def kernel(X_a, edge_index_a, X_b, edge_index_b, W0, b0, W1, b1):
    e = edge_index_a.shape[1]
    e2 = e // 2
    ept = e2 // _NSUB
    n_chunks = 3 * (-(-ept // (_CHUNK * 3)))
    pad = n_chunks * _CHUNK - ept
    pad_src = ((jnp.arange(2 * _NSUB * pad, dtype=jnp.int32) * 97) % _N
               ).reshape(2, _NSUB, pad)
    pad_dst = (_N + (jnp.arange(2 * _NSUB * pad, dtype=jnp.int32)
                     % (_NPAD - _N))).reshape(2, _NSUB, pad)

    def prep(ei):
        src = jnp.concatenate([ei[0].reshape(2, _NSUB, ept), pad_src],
                              axis=2)
        dst = jnp.concatenate([ei[1].reshape(2, _NSUB, ept), pad_dst],
                              axis=2)
        return jnp.stack([src.reshape(2, _NSUB * n_chunks, _CHUNK),
                          dst.reshape(2, _NSUB * n_chunks, _CHUNK)], axis=2)

    idx_a = prep(edge_index_a)
    idx_b = prep(edge_index_b)

    def pad_x(x):
        return jnp.concatenate(
            [x, jnp.zeros((_NPAD - _N, _D), jnp.float32)], axis=0)

    h_a = _tc_mm(pad_x(X_a), W0)
    h_b = _tc_mm(pad_x(X_b), W0)
    s1_a = _spmm_call(h_a, idx_a, n_chunks)
    s1_b = _spmm_call(h_b, idx_b, n_chunks)
    h2_a = _tc_mm2(s1_a, W1, b0)
    h2_b = _tc_mm2(s1_b, W1, b0)
    s2_a = _spmm_call(h2_a, idx_a, n_chunks)
    s2_b = _spmm_call(h2_b, idx_b, n_chunks)
    out_a = _tc_std(s2_a[:, :_N], b1)
    out_b = _tc_std(s2_b[:, :_N], b1)
    return (out_a, out_b)


# speedup vs baseline: 1.2643x; 1.2643x over previous
import functools

import jax
import jax.numpy as jnp
from jax import lax
from jax.experimental import pallas as pl
from jax.experimental.pallas import tpu as pltpu
from jax.experimental.pallas import tpu_sc as plsc

def _gelu(x):
    return 0.5 * x * (1.0 + lax.erf(x * 0.7071067811865476))


_N = 10000
_D = 128
_NPAD = 10112
_NSUB = 16
_CHUNK = 120


def _spmm_call(table, idx_packed, n_chunks):
    rpt = _NPAD // _NSUB
    nt = n_chunks // 6
    n = n_chunks
    npairs = n // 2

    def body(table_hbm, idx_hbm, zero_hbm, out_hbm,
             i0, i1, i2, r0, r1, r2, acc_sh,
             is0, is1, is2, g0, g1, g2, s0, s1, s2):
        c = lax.axis_index("c")
        s = lax.axis_index("s")
        idx = (i0, i1, i2)
        rows = (r0, r1, r2)
        isem = (is0, is1, is2)
        gsem = (g0, g1, g2)
        ssem = (s0, s1, s2)
        pltpu.sync_copy(zero_hbm, acc_sh.at[pl.ds(s * rpt, rpt)])
        plsc.subcore_barrier()

        row0 = s * npairs

        def idx_load(p, up):
            pltpu.async_copy(idx_hbm.at[c, row0 + p], idx[up], isem[up])

        def idx_wait(p, up):
            pltpu.make_async_copy(idx_hbm.at[c, row0 + p], idx[up],
                                  isem[up]).wait()

        def gth(up, q, ur):
            pltpu.async_copy(table_hbm.at[idx[up].at[q, 0]], rows[ur],
                             gsem[ur])

        def gth_wait(up, q, ur):
            pltpu.make_async_copy(table_hbm.at[idx[up].at[q, 0]],
                                  rows[ur], gsem[ur]).wait()

        def scat(up, q, ur):
            pltpu.async_copy(rows[ur], acc_sh.at[idx[up].at[q, 1]],
                             ssem[ur], add=True)

        def scat_wait(up, q, ur):
            pltpu.make_async_copy(rows[ur], acc_sh.at[idx[up].at[q, 1]],
                                  ssem[ur]).wait()

        for p in range(3):
            idx_load(p, p)
        idx_wait(0, 0)
        gth(0, 0, 0)
        gth(0, 1, 1)

        def step(t, carry):
            for u in range(6):
                k = 6 * t + u
                up = (u // 2) % 3
                q = u % 2
                ur = u % 3
                gth_wait(up, q, ur)

                if q == 0:
                    @pl.when(k >= 1)
                    def _():
                        scat_wait((up + 2) % 3, 1, (ur + 2) % 3)

                    @pl.when(k + 2 < n)
                    def _():
                        idx_wait(k // 2 + 1, (up + 1) % 3)
                        gth((up + 1) % 3, 0, (ur + 2) % 3)

                    @pl.when(k + 4 < n)
                    def _():
                        idx_load(k // 2 + 2, (up + 2) % 3)
                else:
                    scat_wait(up, 0, (ur + 2) % 3)

                    @pl.when(k + 2 < n)
                    def _():
                        gth((up + 1) % 3, 1, (ur + 2) % 3)

                scat(up, q, ur)
            return carry

        lax.fori_loop(0, nt, step, 0)
        scat_wait((((n - 1) // 2) % 3), 1, (n - 1) % 3)
        plsc.subcore_barrier()
        pltpu.sync_copy(acc_sh.at[pl.ds(s * rpt, rpt)],
                        out_hbm.at[c, pl.ds(s * rpt, rpt)])

    mesh = plsc.VectorSubcoreMesh(core_axis_name="c", subcore_axis_name="s")
    f = pl.kernel(
        body,
        out_type=jax.ShapeDtypeStruct((2, _NPAD, _D), jnp.float32),
        mesh=mesh,
        scratch_types=[
            pltpu.VMEM((2, 2, _CHUNK), jnp.int32),
            pltpu.VMEM((2, 2, _CHUNK), jnp.int32),
            pltpu.VMEM((2, 2, _CHUNK), jnp.int32),
            pltpu.VMEM((_CHUNK, _D), jnp.float32),
            pltpu.VMEM((_CHUNK, _D), jnp.float32),
            pltpu.VMEM((_CHUNK, _D), jnp.float32),
            pltpu.VMEM_SHARED((_NPAD, _D), jnp.float32),
        ] + [pltpu.SemaphoreType.DMA] * 9,
    )
    zero = jnp.zeros((rpt, _D), jnp.float32)
    return f(table, idx_packed, zero)


def _tc_mm(x, w, b, act):
    m = x.shape[0]
    bm = m // 8
    assert m % 8 == 0 and bm % 8 == 0

    def body(x_ref, w_ref, b_ref, o_ref):
        xv = x_ref[...]
        if act:
            xv = _gelu(xv + b_ref[...])
        o_ref[...] = jnp.dot(xv, w_ref[...], preferred_element_type=jnp.float32)

    return pl.pallas_call(
        body,
        grid=(m // bm,),
        in_specs=[
            pl.BlockSpec((bm, _D), lambda i: (i, 0)),
            pl.BlockSpec((_D, _D), lambda i: (0, 0)),
            pl.BlockSpec((1, _D), lambda i: (0, 0)),
        ],
        out_specs=pl.BlockSpec((bm, _D), lambda i: (i, 0)),
        out_shape=jax.ShapeDtypeStruct((m, _D), jnp.float32),
    )(x, w, b.reshape(1, _D))


def _tc_std(s2, b):
    def body(x_ref, b_ref, o_ref):
        x = x_ref[0] + b_ref[...]
        x = _gelu(x)
        mu = jnp.mean(x, axis=0, keepdims=True)
        xc = x - mu
        var = jnp.sum(xc * xc, axis=0, keepdims=True) / (_N - 1)
        o_ref[0] = xc * lax.rsqrt(var)

    return pl.pallas_call(
        body,
        grid=(2,),
        in_specs=[
            pl.BlockSpec((1, _N, _D), lambda g: (g, 0, 0)),
            pl.BlockSpec((1, _D), lambda g: (0, 0)),
        ],
        out_specs=pl.BlockSpec((1, _N, _D), lambda g: (g, 0, 0)),
        out_shape=jax.ShapeDtypeStruct((2, _N, _D), jnp.float32),
    )(s2, b.reshape(1, _D))


def kernel(X_a, edge_index_a, X_b, edge_index_b, W0, b0, W1, b1):
    e = edge_index_a.shape[1]
    n_chunks = 6 * (-(-e // (_NSUB * _CHUNK * 6)))
    ep = _NSUB * n_chunks * _CHUNK
    pad = ep - e
    pad_src = ((jnp.arange(pad, dtype=jnp.int32) * 97) % _N
               ).reshape(_NSUB, pad // _NSUB)
    pad_dst = (_N + (jnp.arange(pad, dtype=jnp.int32) % (_NPAD - _N))
               ).reshape(_NSUB, pad // _NSUB)

    def prep(ei, coff):
        src = jnp.concatenate([ei[0].reshape(_NSUB, e // _NSUB),
                               pad_src], axis=1) + coff
        dst = jnp.concatenate([ei[1].reshape(_NSUB, e // _NSUB),
                               pad_dst], axis=1)
        return jnp.stack([src.reshape(_NSUB * n_chunks // 2, 2, _CHUNK),
                          dst.reshape(_NSUB * n_chunks // 2, 2, _CHUNK)],
                         axis=2)

    idx = jnp.stack([prep(edge_index_a, 0), prep(edge_index_b, _NPAD)])

    xp = jnp.zeros((2, _NPAD, _D), jnp.float32)
    xp = xp.at[0, :_N].set(X_a).at[1, :_N].set(X_b)

    h = _tc_mm(xp.reshape(2 * _NPAD, _D), W0, b0, act=False)
    s1 = _spmm_call(h, idx, n_chunks)
    h2 = _tc_mm(s1.reshape(2 * _NPAD, _D), W1, b0, act=True)
    s2 = _spmm_call(h2, idx, n_chunks)
    out = _tc_std(s2[:, :_N], b1)
    return (out[0], out[1])
